# Initial kernel scaffold; baseline (speedup 1.0000x reference)
import jax
import jax.numpy as jnp
from jax.experimental import pallas as pl


def kernel(q, rel_pos_emb_table):
    raise NotImplementedError("write your pallas kernel here")


# speedup vs baseline: 1.0374x; 1.0374x over previous
import functools

import jax
import jax.numpy as jnp
from jax import lax
from jax.experimental import pallas as pl
from jax.experimental.pallas import tpu as pltpu
from jax.experimental.pallas import tpu_sc as plsc

NH = 12
T = 256
HD = 64
NROWS = 257
NW = 32
UNITS = NH * 8
IB = T // 8

_mesh = plsc.VectorSubcoreMesh(core_axis_name="c", subcore_axis_name="s")


@functools.partial(
    pl.kernel,
    out_type=jax.ShapeDtypeStruct((NH, T, T, HD), jnp.float32),
    mesh=_mesh,
    scratch_types=[
        pltpu.VMEM((4, 128), jnp.int32),
        pltpu.VMEM((512, HD), jnp.float32),
        pltpu.SemaphoreType.DMA,
        pltpu.SemaphoreType.DMA,
    ],
    compiler_params=pltpu.CompilerParams(use_tc_tiling_on_sc=False),
)
def _rel_pos_sc(table_hbm, out_hbm, idx_v, f_v, gsem, osem):
    wid = lax.axis_index("s") * 2 + lax.axis_index("c")
    for r in range(UNITS // NW):
        u = wid + NW * r
        h = u // 8
        i0 = (u % 8) * IB

        for g in range(4):
            for t in range(8):
                m = g * 128 + t * 16 + lax.iota(jnp.int32, 16)
                row = jnp.clip(383 - m, 0, 256)
                idx_v[g, pl.ds(t * 16, 16)] = row * NH + h

        gathers = [
            pltpu.make_async_copy(
                table_hbm.at[idx_v.at[g]], f_v.at[pl.ds(g * 128, 128)], gsem
            )
            for g in range(4)
        ]
        for c in gathers:
            c.start()
        for c in gathers:
            c.wait()

        def fire(ii, carry):
            i = i0 + ii
            pltpu.make_async_copy(
                f_v.at[pl.ds(255 - i, 256)], out_hbm.at[h, i], osem
            ).start()
            return carry

        lax.fori_loop(0, IB, fire, 0)

        def drain(ii, carry):
            pltpu.make_async_copy(
                f_v.at[pl.ds(0, 256)], out_hbm.at[h, i0], osem
            ).wait()
            return carry

        lax.fori_loop(0, IB, drain, 0)


def kernel(q, rel_pos_emb_table):
    B, n_heads, _, head_dim = q.shape
    table64 = rel_pos_emb_table.reshape(NROWS * NH, HD)
    out = _rel_pos_sc(table64)
    return out.reshape(1, NH, T, T, HD)


# speedup vs baseline: 1.4416x; 1.3896x over previous
import functools

import jax
import jax.numpy as jnp
from jax import lax
from jax.experimental import pallas as pl
from jax.experimental.pallas import tpu as pltpu
from jax.experimental.pallas import tpu_sc as plsc

NH = 12
T = 256
HD = 64
NROWS = 257
HPC = NH // 2

_mesh = plsc.VectorSubcoreMesh(core_axis_name="c", subcore_axis_name="s")


@functools.partial(
    pl.kernel,
    out_type=jax.ShapeDtypeStruct((1, NH, T, T, HD), jnp.float32),
    mesh=_mesh,
    scratch_types=[
        pltpu.VMEM((4, 128), jnp.int32),
        pltpu.VMEM((512, HD), jnp.float32),
        pltpu.VMEM_SHARED((HPC, 512, HD), jnp.float32),
        pltpu.SemaphoreType.DMA,
        pltpu.SemaphoreType.DMA,
        pltpu.SemaphoreType.DMA,
    ],
    compiler_params=pltpu.CompilerParams(use_tc_tiling_on_sc=False),
)
def _rel_pos_sc(table_hbm, out_hbm, idx_v, f_v, strips, gsem, psem, osem):
    core = lax.axis_index("c")
    sub = lax.axis_index("s")
    lane = lax.iota(jnp.int32, 16)

    @pl.when(sub < HPC)
    def _build():
        h = core * HPC + sub
        for g in range(4):
            for t in range(8):
                m = g * 128 + t * 16 + lane
                row = jnp.clip(383 - m, 0, 256)
                idx_v[g, pl.ds(t * 16, 16)] = row * NH + h
        gathers = [
            pltpu.make_async_copy(
                table_hbm.at[idx_v.at[g]], f_v.at[pl.ds(g * 128, 128)], gsem
            )
            for g in range(4)
        ]
        for c in gathers:
            c.start()
        for c in gathers:
            c.wait()
        pltpu.make_async_copy(f_v, strips.at[sub], psem).start()
        pltpu.make_async_copy(f_v, strips.at[sub], psem).wait()

    plsc.subcore_barrier()

    def fire(k, carry):
        i = sub * 16 + k
        for hl in range(HPC):
            pltpu.make_async_copy(
                strips.at[hl, pl.ds(255 - i, 256), :],
                out_hbm.at[0, core * HPC + hl, i],
                osem,
            ).start()
        return carry

    lax.fori_loop(0, 16, fire, 0)

    def drain(k, carry):
        pltpu.make_async_copy(
            strips.at[0, pl.ds(0, 256), :],
            out_hbm.at[0, core * HPC, sub * 16],
            osem,
        ).wait()
        return carry

    lax.fori_loop(0, 16 * HPC, drain, 0)


def kernel(q, rel_pos_emb_table):
    table64 = rel_pos_emb_table.reshape(NROWS * NH, HD)
    return _rel_pos_sc(table64)
